# baseline (speedup 1.0000x reference)
import functools

import jax
import jax.numpy as jnp
import numpy as np
from jax import lax
from jax.experimental import pallas as pl
from jax.experimental.pallas import tpu as pltpu
from jax.experimental.pallas import tpu_sc as plsc

_USE_TEXT_QUERY = True
_USE_TEXT_CANDS = True
_N_CANDS = 5
_TEXT_QUESTION = 0
_TEXT_EMBEDDING = 1
_VISUAL_EMBEDDING = 3


def _prefix_ids() -> list:
    ids = []
    if _USE_TEXT_QUERY:
        ids.append(_TEXT_QUESTION)
    if _USE_TEXT_CANDS:
        ids.extend([_TEXT_EMBEDDING] * _N_CANDS)
    return ids


@functools.lru_cache(maxsize=None)
def _make_sc_fill(L: int, D: int):
    info = plsc.get_sparse_core_info()
    NC, NS = info.num_cores, info.num_subcores
    NW = NC * NS
    assert L % NW == 0 and (L // NW) % 8 == 0
    b_per_w = L // NW
    TILE = 16
    n_blocks = b_per_w // TILE
    prefix = _prefix_ids()
    assert len(prefix) < TILE
    block0_ids = prefix + [_VISUAL_EMBEDDING] * (TILE - len(prefix))
    mesh = plsc.VectorSubcoreMesh(core_axis_name="c", subcore_axis_name="s")

    @functools.partial(
        pl.kernel,
        mesh=mesh,
        out_type=jax.ShapeDtypeStruct((L, 1, D), jnp.float32),
        scratch_types=[
            pltpu.VMEM((TILE, 1, D), jnp.float32),
            pltpu.VMEM((TILE, 1, D), jnp.float32),
            pltpu.SemaphoreType.DMA,
        ],
    )
    def k(table_hbm, out_hbm, buf_v, buf0_v, sem):
        wid = lax.axis_index("s") * NC + lax.axis_index("c")
        base = wid * b_per_w
        vis = table_hbm.at[pl.ds(_VISUAL_EMBEDDING, 1)]

        fills = [
            pltpu.async_copy(vis, buf_v.at[pl.ds(r, 1), 0], sem)
            for r in range(TILE)
        ]

        @pl.when(wid == 0)
        def _fill_prefix():
            f0 = [
                pltpu.async_copy(
                    table_hbm.at[pl.ds(i, 1)], buf0_v.at[pl.ds(r, 1), 0], sem
                )
                for r, i in enumerate(block0_ids)
            ]
            for f in f0:
                f.wait()

        for f in fills:
            f.wait()

        @pl.when(wid == 0)
        def _write_block0():
            pltpu.sync_copy(buf0_v, out_hbm.at[pl.ds(base, TILE)])

        @pl.when(wid != 0)
        def _write_block0_common():
            pltpu.sync_copy(buf_v, out_hbm.at[pl.ds(base, TILE)])

        writes = [
            pltpu.async_copy(
                buf_v, out_hbm.at[pl.ds(base + b * TILE, TILE)], sem
            )
            for b in range(1, n_blocks)
        ]
        for w in writes:
            w.wait()

    return k


def kernel(x, table):
    L, N, D = x.shape
    return _make_sc_fill(L, D)(table)


# speedup vs baseline: 1.8661x; 1.8661x over previous
import functools

import jax
import jax.numpy as jnp
import numpy as np
from jax import lax
from jax.experimental import pallas as pl
from jax.experimental.pallas import tpu as pltpu
from jax.experimental.pallas import tpu_sc as plsc

_USE_TEXT_QUERY = True
_USE_TEXT_CANDS = True
_N_CANDS = 5
_TEXT_QUESTION = 0
_TEXT_EMBEDDING = 1
_VISUAL_EMBEDDING = 3
_PREFIX_ROWS = 16


@functools.lru_cache(maxsize=None)
def _make_sc_prefix(D: int):
    info = plsc.get_sparse_core_info()
    NC = info.num_cores
    mesh = plsc.VectorSubcoreMesh(core_axis_name="c", subcore_axis_name="s")
    n_text = (1 if _USE_TEXT_QUERY else 0) + (
        _N_CANDS if _USE_TEXT_CANDS else 0
    )

    @functools.partial(
        pl.kernel,
        mesh=mesh,
        out_type=jax.ShapeDtypeStruct((_PREFIX_ROWS, D), jnp.float32),
        scratch_types=[
            pltpu.VMEM((_PREFIX_ROWS, D), jnp.float32),
            pltpu.SemaphoreType.DMA,
        ],
    )
    def k(table_hbm, out_hbm, rows_v, sem):
        wid = lax.axis_index("s") * NC + lax.axis_index("c")

        @pl.when(wid == 0)
        def _gather():
            lane = lax.iota(jnp.int32, 16)
            ids = jnp.where(
                lane < 1,
                _TEXT_QUESTION,
                jnp.where(lane < n_text, _TEXT_EMBEDDING, _VISUAL_EMBEDDING),
            )
            pltpu.async_copy(table_hbm.at[ids], rows_v, sem).wait()
            pltpu.sync_copy(rows_v, out_hbm)

    return k


@functools.lru_cache(maxsize=None)
def _make_tc_fill(L: int, D: int):
    BLK = 256
    assert L % BLK == 0

    def body(table_ref, out_ref):
        row = table_ref[_VISUAL_EMBEDDING, :]
        out_ref[...] = jnp.broadcast_to(row[None, None, :], (BLK, 1, D))

    return pl.pallas_call(
        body,
        grid=(L // BLK,),
        in_specs=[pl.BlockSpec((5, D), lambda i: (0, 0))],
        out_specs=pl.BlockSpec((BLK, 1, D), lambda i: (i, 0, 0)),
        out_shape=jax.ShapeDtypeStruct((L, 1, D), jnp.float32),
    )


def kernel(x, table):
    L, N, D = x.shape
    prefix = _make_sc_prefix(D)(table)
    full = _make_tc_fill(L, D)(table)
    return lax.dynamic_update_slice(
        full, prefix.reshape(_PREFIX_ROWS, 1, D), (0, 0, 0)
    )


# speedup vs baseline: 1.9964x; 1.0698x over previous
import functools

import jax
import jax.numpy as jnp
import numpy as np
from jax import lax
from jax.experimental import pallas as pl
from jax.experimental.pallas import tpu as pltpu
from jax.experimental.pallas import tpu_sc as plsc

_USE_TEXT_QUERY = True
_USE_TEXT_CANDS = True
_N_CANDS = 5
_TEXT_QUESTION = 0
_TEXT_EMBEDDING = 1
_VISUAL_EMBEDDING = 3
_PREFIX_ROWS = 16


@functools.lru_cache(maxsize=None)
def _make_sc_prefix(D: int):
    info = plsc.get_sparse_core_info()
    NC = info.num_cores
    mesh = plsc.VectorSubcoreMesh(
        core_axis_name="c", subcore_axis_name="s", num_cores=1
    )
    n_text = (1 if _USE_TEXT_QUERY else 0) + (
        _N_CANDS if _USE_TEXT_CANDS else 0
    )

    @functools.partial(
        pl.kernel,
        mesh=mesh,
        out_type=jax.ShapeDtypeStruct((_PREFIX_ROWS, D), jnp.float32),
        scratch_types=[
            pltpu.VMEM((_PREFIX_ROWS, D), jnp.float32),
            pltpu.SemaphoreType.DMA,
        ],
    )
    def k(table_hbm, out_hbm, rows_v, sem):
        wid = lax.axis_index("s") * NC + lax.axis_index("c")

        @pl.when(wid == 0)
        def _gather():
            lane = lax.iota(jnp.int32, 16)
            ids = jnp.where(
                lane < 1,
                _TEXT_QUESTION,
                jnp.where(lane < n_text, _TEXT_EMBEDDING, _VISUAL_EMBEDDING),
            )
            pltpu.async_copy(table_hbm.at[ids], rows_v, sem).wait()
            pltpu.sync_copy(rows_v, out_hbm)

    return k


@functools.lru_cache(maxsize=None)
def _make_tc_fill(L: int, D: int):
    BLK = 256
    assert L % BLK == 0

    def body(table_ref, out_ref):
        row = table_ref[_VISUAL_EMBEDDING, :]
        out_ref[...] = jnp.broadcast_to(row[None, None, :], (BLK, 1, D))

    return pl.pallas_call(
        body,
        grid=(L // BLK,),
        in_specs=[pl.BlockSpec((5, D), lambda i: (0, 0))],
        out_specs=pl.BlockSpec((BLK, 1, D), lambda i: (i, 0, 0)),
        out_shape=jax.ShapeDtypeStruct((L, 1, D), jnp.float32),
    )


def kernel(x, table):
    L, N, D = x.shape
    prefix = _make_sc_prefix(D)(table)
    full = _make_tc_fill(L, D)(table)
    return lax.dynamic_update_slice(
        full, prefix.reshape(_PREFIX_ROWS, 1, D), (0, 0, 0)
    )
